# speedup vs baseline: 6.7118x; 6.7118x over previous
import jax
import jax.numpy as jnp
from jax.experimental import pallas as pl
from jax.experimental.pallas import tpu as pltpu

V = 32000
D = 1024
NL = 2
DI = 2 * D
N = 16
DTR = D // 16
KW = 4
OUT = 32000
B, L = 4, 1024

TT = 256
C = 256
HT = 3200
F32 = jnp.float32
BF16 = jnp.bfloat16


def _embed_body(ids_ref, emb_ref, nw_ref, win_ref, x0_ref, xz_ref, xg, sem):
    i = pl.program_id(0)
    base = i * TT
    cps = []
    for mi in range(TT):
        cp = pltpu.make_async_copy(emb_ref.at[ids_ref[base + mi]], xg.at[mi], sem)
        cp.start()
        cps.append(cp)
    for cp in cps:
        cp.wait()
    x = xg[...]
    x0_ref[...] = x
    xn = x * jax.lax.rsqrt(jnp.mean(x * x, axis=-1, keepdims=True) + 1e-5) * nw_ref[...]
    xz_ref[...] = jnp.dot(xn.astype(BF16), win_ref[...],
                          preferred_element_type=F32).astype(BF16)


def _embed(ids, emb, nw, win):
    nt = (B * L) // TT
    return pl.pallas_call(
        _embed_body,
        grid_spec=pltpu.PrefetchScalarGridSpec(
            num_scalar_prefetch=1,
            grid=(nt,),
            in_specs=[
                pl.BlockSpec(memory_space=pl.ANY),
                pl.BlockSpec((1, D), lambda i, ids: (0, 0)),
                pl.BlockSpec((D, 2 * DI), lambda i, ids: (0, 0)),
            ],
            out_specs=[
                pl.BlockSpec((TT, D), lambda i, ids: (i, 0)),
                pl.BlockSpec((TT, 2 * DI), lambda i, ids: (i, 0)),
            ],
            scratch_shapes=[
                pltpu.VMEM((TT, D), F32),
                pltpu.SemaphoreType.DMA,
            ],
        ),
        out_shape=[
            jax.ShapeDtypeStruct((B * L, D), F32),
            jax.ShapeDtypeStruct((B * L, 2 * DI), BF16),
        ],
        compiler_params=pltpu.CompilerParams(
            dimension_semantics=("parallel",),
            vmem_limit_bytes=48 * 1024 * 1024,
        ),
        name="embed_in0",
    )(ids, emb, nw, win)


def _mixer_body(xz_ref, cw_ref, cb_ref, xpw_ref, dtw_ref, dtb_ref, alog_ref,
                dsk_ref, y_ref, xp, xc3, dl3, db3, y3, st, cvc, As):
    j = pl.program_id(1)

    @pl.when(j == 0)
    def _():
        cvc[...] = jnp.zeros_like(cvc)
        st[...] = jnp.zeros_like(st)
        As[...] = -jnp.exp(alog_ref[...])

    xi = xz_ref[:, :DI].astype(F32)
    xp[0:8, :] = cvc[...]
    xp[8:, :] = xi
    cvc[...] = xp[C:C + 8, :]

    acc = (cb_ref[...]
           + cw_ref[0:1, :] * xp[5:5 + C, :]
           + cw_ref[1:2, :] * xp[6:6 + C, :]
           + cw_ref[2:3, :] * xp[7:7 + C, :]
           + cw_ref[3:4, :] * xp[8:8 + C, :])
    xcv = acc * jax.nn.sigmoid(acc)
    xc3[...] = xcv.reshape(C, 1, DI)

    dbl = jnp.dot(xcv.astype(BF16), xpw_ref[...], preferred_element_type=F32)
    db3[...] = dbl.reshape(C, 1, DTR + 2 * N)
    dlin = jnp.dot(dbl[:, :DTR].astype(BF16), dtw_ref[...],
                   preferred_element_type=F32) + dtb_ref[...]
    dlt = jnp.where(dlin > 20.0, dlin,
                    jnp.log1p(jnp.exp(jnp.minimum(dlin, 20.0))))
    dl3[...] = dlt.reshape(C, 1, DI)

    niota = jax.lax.broadcasted_iota(jnp.int32, (N, DTR + 2 * N), 0)
    liota = jax.lax.broadcasted_iota(jnp.int32, (N, DTR + 2 * N), 1)
    mb = liota == (DTR + niota)
    mc = liota == (DTR + N + niota)

    U = 8

    def step(g, _):
        for k in range(U):
            t = g * U + k
            dtr = dl3[pl.ds(t, 1)].reshape(1, DI)
            xcr = xc3[pl.ds(t, 1)].reshape(1, DI)
            bcr = jnp.broadcast_to(db3[pl.ds(t, 1)].reshape(1, DTR + 2 * N),
                                   (N, DTR + 2 * N))
            bcol = jnp.sum(jnp.where(mb, bcr, 0.0), axis=1, keepdims=True)
            ccol = jnp.sum(jnp.where(mc, bcr, 0.0), axis=1, keepdims=True)
            dA = jnp.exp(As[...] * dtr)
            s_new = dA * st[...] + (dtr * xcr) * bcol
            st[...] = s_new
            y3[pl.ds(t, 1)] = jnp.sum(s_new * ccol, axis=0,
                                      keepdims=True).reshape(1, 1, DI)
        return 0

    jax.lax.fori_loop(0, C // U, step, 0)

    z = xz_ref[:, DI:].astype(F32)
    ys = y3[...].reshape(C, DI)
    yv = (ys + dsk_ref[...] * xcv) * (z * jax.nn.sigmoid(z))
    y_ref[...] = yv.astype(y_ref.dtype)


def _mixer(xz, cw, cb, xpw, dtw, dtb, alogT, dsk, lname):
    nj = L // C
    return pl.pallas_call(
        _mixer_body,
        grid=(B, nj),
        in_specs=[
            pl.BlockSpec((C, 2 * DI), lambda b, j: (b * (L // C) + j, 0)),
            pl.BlockSpec((KW, DI), lambda b, j: (0, 0)),
            pl.BlockSpec((1, DI), lambda b, j: (0, 0)),
            pl.BlockSpec((DI, DTR + 2 * N), lambda b, j: (0, 0)),
            pl.BlockSpec((DTR, DI), lambda b, j: (0, 0)),
            pl.BlockSpec((1, DI), lambda b, j: (0, 0)),
            pl.BlockSpec((N, DI), lambda b, j: (0, 0)),
            pl.BlockSpec((1, DI), lambda b, j: (0, 0)),
        ],
        out_specs=pl.BlockSpec((C, DI), lambda b, j: (b * (L // C) + j, 0)),
        out_shape=jax.ShapeDtypeStruct((B * L, DI), BF16),
        scratch_shapes=[
            pltpu.VMEM((C + 8, DI), F32),
            pltpu.VMEM((C, 1, DI), F32),
            pltpu.VMEM((C, 1, DI), F32),
            pltpu.VMEM((C, 1, DTR + 2 * N), F32),
            pltpu.VMEM((C, 1, DI), F32),
            pltpu.VMEM((N, DI), F32),
            pltpu.VMEM((8, DI), F32),
            pltpu.VMEM((N, DI), F32),
        ],
        compiler_params=pltpu.CompilerParams(
            dimension_semantics=("parallel", "arbitrary"),
            vmem_limit_bytes=48 * 1024 * 1024,
        ),
        name=lname,
    )(xz, cw, cb, xpw, dtw, dtb, alogT, dsk)


def _mid_body(x0_ref, y0_ref, wout_ref, nw_ref, win_ref, x1_ref, xz_ref):
    x1 = x0_ref[...] + jnp.dot(y0_ref[...], wout_ref[...],
                               preferred_element_type=F32)
    x1_ref[...] = x1
    xn = x1 * jax.lax.rsqrt(jnp.mean(x1 * x1, axis=-1, keepdims=True) + 1e-5) * nw_ref[...]
    xz_ref[...] = jnp.dot(xn.astype(BF16), win_ref[...],
                          preferred_element_type=F32).astype(BF16)


def _mid(x0, y0, wout, nw, win):
    nt = (B * L) // TT
    return pl.pallas_call(
        _mid_body,
        grid=(nt,),
        in_specs=[
            pl.BlockSpec((TT, D), lambda i: (i, 0)),
            pl.BlockSpec((TT, DI), lambda i: (i, 0)),
            pl.BlockSpec((DI, D), lambda i: (0, 0)),
            pl.BlockSpec((1, D), lambda i: (0, 0)),
            pl.BlockSpec((D, 2 * DI), lambda i: (0, 0)),
        ],
        out_specs=[
            pl.BlockSpec((TT, D), lambda i: (i, 0)),
            pl.BlockSpec((TT, 2 * DI), lambda i: (i, 0)),
        ],
        out_shape=[
            jax.ShapeDtypeStruct((B * L, D), F32),
            jax.ShapeDtypeStruct((B * L, 2 * DI), BF16),
        ],
        compiler_params=pltpu.CompilerParams(
            dimension_semantics=("parallel",),
            vmem_limit_bytes=48 * 1024 * 1024,
        ),
        name="mid_out0_in1",
    )(x0, y0, wout, nw, win)


def _final_body(x1_ref, y1_ref, wout_ref, nw_ref, hn_ref):
    h = x1_ref[...] + jnp.dot(y1_ref[...], wout_ref[...],
                              preferred_element_type=F32)
    hn = h * jax.lax.rsqrt(jnp.mean(h * h, axis=-1, keepdims=True) + 1e-5) * nw_ref[...]
    hn_ref[...] = hn.astype(BF16)


def _final(x1, y1, wout, nw):
    nt = (B * L) // TT
    return pl.pallas_call(
        _final_body,
        grid=(nt,),
        in_specs=[
            pl.BlockSpec((TT, D), lambda i: (i, 0)),
            pl.BlockSpec((TT, DI), lambda i: (i, 0)),
            pl.BlockSpec((DI, D), lambda i: (0, 0)),
            pl.BlockSpec((1, D), lambda i: (0, 0)),
        ],
        out_specs=pl.BlockSpec((TT, D), lambda i: (i, 0)),
        out_shape=jax.ShapeDtypeStruct((B * L, D), BF16),
        compiler_params=pltpu.CompilerParams(
            dimension_semantics=("parallel",),
            vmem_limit_bytes=48 * 1024 * 1024,
        ),
        name="final_norm",
    )(x1, y1, wout, nw)


def _head_body(x_ref, w_ref, o_ref):
    o_ref[...] = jax.lax.dot_general(x_ref[...], w_ref[...],
                                     (((1,), (1,)), ((), ())),
                                     preferred_element_type=F32)


def _head(hn, w):
    nt = (B * L) // TT
    no = OUT // HT
    return pl.pallas_call(
        _head_body,
        grid=(no, nt),
        in_specs=[
            pl.BlockSpec((TT, D), lambda i, j: (j, 0)),
            pl.BlockSpec((HT, D), lambda i, j: (i, 0)),
        ],
        out_specs=pl.BlockSpec((TT, HT), lambda i, j: (j, i)),
        out_shape=jax.ShapeDtypeStruct((B * L, OUT), F32),
        compiler_params=pltpu.CompilerParams(
            dimension_semantics=("parallel", "arbitrary"),
            vmem_limit_bytes=48 * 1024 * 1024,
        ),
        name="head",
    )(hn, w)


def kernel(input_ids, emb, norm_w, in_proj, conv_w, conv_b, x_proj, dt_w, dt_b,
           A_log, D_skip, out_proj, final_norm_w, head_w):
    ids = input_ids.reshape(-1).astype(jnp.int32)

    x0, xz0 = _embed(ids, emb, norm_w[0].reshape(1, D),
                     in_proj[0].astype(BF16))
    y0 = _mixer(xz0, conv_w[0].T, conv_b[0].reshape(1, DI),
                x_proj[0].astype(BF16), dt_w[0].astype(BF16),
                dt_b[0].reshape(1, DI), A_log[0].T, D_skip[0].reshape(1, DI),
                "mixer0")
    x1, xz1 = _mid(x0, y0, out_proj[0].astype(BF16), norm_w[1].reshape(1, D),
                   in_proj[1].astype(BF16))
    y1 = _mixer(xz1, conv_w[1].T, conv_b[1].reshape(1, DI),
                x_proj[1].astype(BF16), dt_w[1].astype(BF16),
                dt_b[1].reshape(1, DI), A_log[1].T, D_skip[1].reshape(1, DI),
                "mixer1")
    hn = _final(x1, y1, out_proj[1].astype(BF16), final_norm_w.reshape(1, D))
    logits = _head(hn, head_w.astype(BF16))
    return logits.reshape(B, L, OUT)


# speedup vs baseline: 7.8970x; 1.1766x over previous
import jax
import jax.numpy as jnp
from jax.experimental import pallas as pl
from jax.experimental.pallas import tpu as pltpu

V = 32000
D = 1024
NL = 2
DI = 2 * D
N = 16
DTR = D // 16
KW = 4
OUT = 32000
B, L = 4, 1024

TT = 256
C = 256
U = 8
HT = 3200
F32 = jnp.float32
BF16 = jnp.bfloat16


def _rmsnorm(x, w):
    return x * jax.lax.rsqrt(jnp.mean(x * x, axis=-1, keepdims=True) + 1e-5) * w


def _mamba_core(xz, j, cw_ref, cb_ref, xpw_ref, dtw_ref, dtb_ref, alog_ref,
                dsk_ref, y_ref, xp, xc3, dl3, db3, y3, st, cvc, As, q8, u3, zs):
    @pl.when(j == 0)
    def _():
        cvc[...] = jnp.zeros_like(cvc)
        st[...] = jnp.zeros_like(st)
        As[...] = -jnp.exp(alog_ref[...])

    zs[...] = xz[:, DI:]
    xi = xz[:, :DI]
    xp[0:8, :] = cvc[...]
    xp[8:, :] = xi
    cvc[...] = xp[C:C + 8, :]

    acc = (cb_ref[...]
           + cw_ref[0:1, :] * xp[5:5 + C, :]
           + cw_ref[1:2, :] * xp[6:6 + C, :]
           + cw_ref[2:3, :] * xp[7:7 + C, :]
           + cw_ref[3:4, :] * xp[8:8 + C, :])
    xcv = acc * jax.nn.sigmoid(acc)
    xc3[...] = xcv

    dbl = jnp.dot(xcv.astype(BF16), xpw_ref[...], preferred_element_type=F32)
    db3[...] = dbl
    dlin = jnp.dot(dbl[:, :DTR].astype(BF16), dtw_ref[...],
                   preferred_element_type=F32) + dtb_ref[...]
    dlt = jnp.where(dlin > 20.0, dlin,
                    jnp.log1p(jnp.exp(jnp.minimum(dlin, 20.0))))
    dl3[...] = dlt

    niota = jax.lax.broadcasted_iota(jnp.int32, (N, DTR + 2 * N), 0)
    liota = jax.lax.broadcasted_iota(jnp.int32, (N, DTR + 2 * N), 1)
    mb = liota == (DTR + niota)
    mc = liota == (DTR + N + niota)
    rio = jax.lax.broadcasted_iota(jnp.int32, (U, U * N), 0)
    cio = jax.lax.broadcasted_iota(jnp.int32, (U, U * N), 1)
    selw = jnp.where((cio // N) == rio, 1.0, 0.0).astype(BF16)

    u3[...] = dlt * xcv

    def group(g, par):
        for k in range(U):
            t = g * U + k
            dtr = dl3[pl.ds(t, 1), :]
            ur = u3[pl.ds(t, 1), :]
            bcr = jnp.broadcast_to(db3[pl.ds(t, 1), :], (N, DTR + 2 * N))
            bcol = jnp.sum(jnp.where(mb, bcr, 0.0), axis=1, keepdims=True)
            ccol = jnp.sum(jnp.where(mc, bcr, 0.0), axis=1, keepdims=True)
            dA = jnp.exp(As[...] * dtr)
            s_new = dA * st[...] + ur * bcol
            st[...] = s_new
            q8[par, k * N:(k + 1) * N, :] = (s_new * ccol).astype(BF16)

    def flushy(g, par):
        y8 = jnp.dot(selw, q8[par], preferred_element_type=F32)
        y3[pl.ds(pl.multiple_of(g * U, U), U), :] = y8

    def step(h, _):
        g0 = h * 2
        group(g0, 0)
        flushy(g0, 0)
        group(g0 + 1, 1)
        flushy(g0 + 1, 1)
        return 0

    jax.lax.fori_loop(0, C // (2 * U), step, 0)

    z = zs[...]
    yv = (y3[...] + dsk_ref[...] * xc3[...]) * (z * jax.nn.sigmoid(z))
    y_ref[...] = yv.astype(y_ref.dtype)


def _mixer_scratch():
    return [
        pltpu.VMEM((C + 8, DI), F32),
        pltpu.VMEM((C, DI), F32),
        pltpu.VMEM((C, DI), F32),
        pltpu.VMEM((C, DTR + 2 * N), F32),
        pltpu.VMEM((C, DI), F32),
        pltpu.VMEM((N, DI), F32),
        pltpu.VMEM((8, DI), F32),
        pltpu.VMEM((N, DI), F32),
        pltpu.VMEM((2, U * N, DI), BF16),
        pltpu.VMEM((C, DI), F32),
        pltpu.VMEM((C, DI), F32),
    ]


def _m0_body(ids_ref, emb_ref, nw_ref, win_ref, cw_ref, cb_ref, xpw_ref,
             dtw_ref, dtb_ref, alog_ref, dsk_ref, x0_ref, y_ref,
             xg, sem, xp, xc3, dl3, db3, y3, st, cvc, As, q8, u3, zs):
    b = pl.program_id(0)
    j = pl.program_id(1)
    base = (b * pl.num_programs(1) + j) * TT
    cps = []
    for mi in range(TT):
        cp = pltpu.make_async_copy(emb_ref.at[ids_ref[base + mi]], xg.at[mi], sem)
        cp.start()
        cps.append(cp)
    for cp in cps:
        cp.wait()
    x = xg[...]
    x0_ref[...] = x
    xz = jnp.dot(_rmsnorm(x, nw_ref[...]).astype(BF16), win_ref[...],
                 preferred_element_type=F32)
    _mamba_core(xz, j, cw_ref, cb_ref, xpw_ref, dtw_ref, dtb_ref, alog_ref,
                dsk_ref, y_ref, xp, xc3, dl3, db3, y3, st, cvc, As, q8, u3, zs)


def _m0(ids, emb, nw, win, cw, cb, xpw, dtw, dtb, alogT, dsk):
    nj = L // C
    wspec = lambda shape: pl.BlockSpec(shape, lambda b, j, ids: (0, 0))
    return pl.pallas_call(
        _m0_body,
        grid_spec=pltpu.PrefetchScalarGridSpec(
            num_scalar_prefetch=1,
            grid=(B, nj),
            in_specs=[
                pl.BlockSpec(memory_space=pl.ANY),
                wspec((1, D)),
                wspec((D, 2 * DI)),
                wspec((KW, DI)),
                wspec((1, DI)),
                wspec((DI, DTR + 2 * N)),
                wspec((DTR, DI)),
                wspec((1, DI)),
                wspec((N, DI)),
                wspec((1, DI)),
            ],
            out_specs=[
                pl.BlockSpec((TT, D), lambda b, j, ids: (b * (L // C) + j, 0)),
                pl.BlockSpec((C, DI), lambda b, j, ids: (b * (L // C) + j, 0)),
            ],
            scratch_shapes=[pltpu.VMEM((TT, D), F32), pltpu.SemaphoreType.DMA]
            + _mixer_scratch(),
        ),
        out_shape=[
            jax.ShapeDtypeStruct((B * L, D), F32),
            jax.ShapeDtypeStruct((B * L, DI), BF16),
        ],
        compiler_params=pltpu.CompilerParams(
            dimension_semantics=("parallel", "arbitrary"),
            vmem_limit_bytes=48 * 1024 * 1024,
        ),
        name="m0_embed_mixer",
    )(ids, emb, nw, win, cw, cb, xpw, dtw, dtb, alogT, dsk)


def _m1_body(x0_ref, y0_ref, wout_ref, nw_ref, win_ref, cw_ref, cb_ref,
             xpw_ref, dtw_ref, dtb_ref, alog_ref, dsk_ref, x1_ref, y_ref,
             xp, xc3, dl3, db3, y3, st, cvc, As, q8, u3, zs):
    j = pl.program_id(1)
    x1 = x0_ref[...] + jnp.dot(y0_ref[...], wout_ref[...],
                               preferred_element_type=F32)
    x1_ref[...] = x1
    xz = jnp.dot(_rmsnorm(x1, nw_ref[...]).astype(BF16), win_ref[...],
                 preferred_element_type=F32)
    _mamba_core(xz, j, cw_ref, cb_ref, xpw_ref, dtw_ref, dtb_ref, alog_ref,
                dsk_ref, y_ref, xp, xc3, dl3, db3, y3, st, cvc, As, q8, u3, zs)


def _m1(x0, y0, wout, nw, win, cw, cb, xpw, dtw, dtb, alogT, dsk):
    nj = L // C
    tspec = lambda w: pl.BlockSpec((C, w), lambda b, j: (b * (L // C) + j, 0))
    wspec = lambda shape: pl.BlockSpec(shape, lambda b, j: (0, 0))
    return pl.pallas_call(
        _m1_body,
        grid=(B, nj),
        in_specs=[
            tspec(D),
            tspec(DI),
            wspec((DI, D)),
            wspec((1, D)),
            wspec((D, 2 * DI)),
            wspec((KW, DI)),
            wspec((1, DI)),
            wspec((DI, DTR + 2 * N)),
            wspec((DTR, DI)),
            wspec((1, DI)),
            wspec((N, DI)),
            wspec((1, DI)),
        ],
        out_specs=[
            pl.BlockSpec((TT, D), lambda b, j: (b * (L // C) + j, 0)),
            pl.BlockSpec((C, DI), lambda b, j: (b * (L // C) + j, 0)),
        ],
        out_shape=[
            jax.ShapeDtypeStruct((B * L, D), F32),
            jax.ShapeDtypeStruct((B * L, DI), BF16),
        ],
        scratch_shapes=_mixer_scratch(),
        compiler_params=pltpu.CompilerParams(
            dimension_semantics=("parallel", "arbitrary"),
            vmem_limit_bytes=48 * 1024 * 1024,
        ),
        name="m1_mid_mixer",
    )(x0, y0, wout, nw, win, cw, cb, xpw, dtw, dtb, alogT, dsk)


def _final_body(x1_ref, y1_ref, wout_ref, nw_ref, hn_ref):
    h = x1_ref[...] + jnp.dot(y1_ref[...], wout_ref[...],
                              preferred_element_type=F32)
    hn_ref[...] = _rmsnorm(h, nw_ref[...]).astype(BF16)


def _final(x1, y1, wout, nw):
    nt = (B * L) // TT
    return pl.pallas_call(
        _final_body,
        grid=(nt,),
        in_specs=[
            pl.BlockSpec((TT, D), lambda i: (i, 0)),
            pl.BlockSpec((TT, DI), lambda i: (i, 0)),
            pl.BlockSpec((DI, D), lambda i: (0, 0)),
            pl.BlockSpec((1, D), lambda i: (0, 0)),
        ],
        out_specs=pl.BlockSpec((TT, D), lambda i: (i, 0)),
        out_shape=jax.ShapeDtypeStruct((B * L, D), BF16),
        compiler_params=pltpu.CompilerParams(
            dimension_semantics=("parallel",),
            vmem_limit_bytes=48 * 1024 * 1024,
        ),
        name="final_norm",
    )(x1, y1, wout, nw)


def _head_body(x_ref, w_ref, o_ref):
    o_ref[...] = jnp.dot(x_ref[...], w_ref[...], preferred_element_type=F32)


def _head(hn, w):
    HM = 512
    nt = (B * L) // HM
    no = OUT // HT
    return pl.pallas_call(
        _head_body,
        grid=(no, nt),
        in_specs=[
            pl.BlockSpec((HM, D), lambda i, j: (j, 0)),
            pl.BlockSpec((D, HT), lambda i, j: (0, i)),
        ],
        out_specs=pl.BlockSpec((HM, HT), lambda i, j: (j, i)),
        out_shape=jax.ShapeDtypeStruct((B * L, OUT), F32),
        compiler_params=pltpu.CompilerParams(
            dimension_semantics=("parallel", "arbitrary"),
            vmem_limit_bytes=48 * 1024 * 1024,
        ),
        name="head",
    )(hn, w)


def kernel(input_ids, emb, norm_w, in_proj, conv_w, conv_b, x_proj, dt_w, dt_b,
           A_log, D_skip, out_proj, final_norm_w, head_w):
    ids = input_ids.reshape(-1).astype(jnp.int32)

    x0, y0 = _m0(ids, emb, norm_w[0].reshape(1, D), in_proj[0].astype(BF16),
                 conv_w[0].T, conv_b[0].reshape(1, DI), x_proj[0].astype(BF16),
                 dt_w[0].astype(BF16), dt_b[0].reshape(1, DI), A_log[0].T,
                 D_skip[0].reshape(1, DI))
    x1, y1 = _m1(x0, y0, out_proj[0].astype(BF16), norm_w[1].reshape(1, D),
                 in_proj[1].astype(BF16), conv_w[1].T, conv_b[1].reshape(1, DI),
                 x_proj[1].astype(BF16), dt_w[1].astype(BF16),
                 dt_b[1].reshape(1, DI), A_log[1].T, D_skip[1].reshape(1, DI))
    hn = _final(x1, y1, out_proj[1].astype(BF16), final_norm_w.reshape(1, D))
    logits = _head(hn, head_w.T.astype(BF16))
    return logits.reshape(B, L, OUT)
